# baseline (speedup 1.0000x reference)
import functools

import jax
import jax.numpy as jnp
from jax import lax
from jax.experimental import pallas as pl
from jax.experimental.pallas import tpu as pltpu
from jax.experimental.pallas import tpu_sc as plsc

NSPLITS = 8
ALIGN = 128
LANES = 16
CH = 64
GAP_WPG = 4
NW = 32


def _make_copy_rows(d):
    def _copy_rows(src, dst, slo, dlo, cnt):
        nfull = cnt // CH

        def body(j, carry):
            o = j * CH
            s = pl.multiple_of((slo + o) * d, d)
            t = pl.multiple_of((dlo + o) * d, d)
            pltpu.sync_copy(src.at[pl.ds(s, CH * d)], dst.at[pl.ds(t, CH * d)])
            return carry

        lax.fori_loop(0, nfull, body, 0)
        off = nfull * CH
        b = CH // 2
        while b >= 1:
            bit = b
            o = off

            @pl.when((cnt & bit) != 0)
            def _():
                s = pl.multiple_of((slo + o) * d, d)
                t = pl.multiple_of((dlo + o) * d, d)
                pltpu.sync_copy(src.at[pl.ds(s, bit * d)],
                                dst.at[pl.ds(t, bit * d)])

            off = off + jnp.where((cnt & bit) != 0, bit, 0)
            b //= 2

    return _copy_rows


@functools.cache
def _make_dispatch(in_len, out_len, d):
    rows_per_w = in_len // NW
    copy_rows = _make_copy_rows(d)
    mesh = plsc.VectorSubcoreMesh(core_axis_name="c", subcore_axis_name="s")

    @functools.partial(
        pl.kernel,
        out_type=jax.ShapeDtypeStruct((out_len * d,), jnp.float32),
        mesh=mesh,
        scratch_types=[pltpu.VMEM((LANES,), jnp.int32)],
    )
    def dispatch(inp_h, out_h, splits_h, res_h, splits_v):
        wid = lax.axis_index("s") * 2 + lax.axis_index("c")
        pltpu.sync_copy(splits_h, splits_v)
        sv = splits_v[...]

        splits, starts, ends, offs, offs_end, shifts = [], [], [], [], [], []
        end_acc = jnp.int32(0)
        off_acc = jnp.int32(0)
        for e in range(NSPLITS):
            s = sv[e]
            splits.append(s)
            starts.append(end_acc)
            end_acc = end_acc + s
            ends.append(end_acc)
            offs.append(off_acc)
            off_acc = off_acc + ((s + (ALIGN - 1)) & jnp.int32(-ALIGN))
            offs_end.append(off_acc)
            shifts.append(offs[e] - starts[e])

        wlo = wid * rows_per_w
        whi = wlo + rows_per_w
        for e in range(NSPLITS):
            lo = jnp.maximum(starts[e], wlo)
            hi = jnp.minimum(ends[e], whi)
            cnt = jnp.maximum(hi - lo, 0)
            copy_rows(inp_h, res_h, lo, lo + shifts[e], cnt)

        g = wid % NSPLITS
        q = wid // NSPLITS
        gs = jnp.int32(0)
        gend = jnp.int32(out_len)
        for e in range(NSPLITS):
            gs = jnp.where(g == e, offs[e] + splits[e], gs)
            if e < NSPLITS - 1:
                gend = jnp.where(g == e, offs_end[e], gend)
        glen = jnp.maximum(gend - gs, 0)
        qlen = (glen + GAP_WPG - 1) // GAP_WPG
        mylo = gs + q * qlen
        mycnt = jnp.clip(glen - q * qlen, 0, qlen)
        copy_rows(out_h, res_h, mylo, mylo, mycnt)

    return dispatch


def kernel(inp, out, in_splits, out_splits_offsets):
    splits16 = jnp.zeros((LANES,), jnp.int32).at[:NSPLITS].set(
        in_splits.astype(jnp.int32))
    n, d = inp.shape
    m = out.shape[0]
    f = _make_dispatch(n, m, d)
    res = f(inp.reshape(-1), out.reshape(-1), splits16)
    return res.reshape(m, d)
